# Initial kernel scaffold; baseline (speedup 1.0000x reference)
import jax
import jax.numpy as jnp
from jax.experimental import pallas as pl


def kernel(x, adj, W1, b1, W2, b2, W3, b3):
    raise NotImplementedError("write your pallas kernel here")


# speedup vs baseline: 1.0173x; 1.0173x over previous
import jax
import jax.numpy as jnp
from jax.experimental import pallas as pl
from jax.experimental.pallas import tpu as pltpu

_BF = jnp.bfloat16
_F32 = jnp.float32


def _pick_bm(n: int, cap: int = 256) -> int:
    best = 8
    for bm in range(8, min(n, cap) + 1, 8):
        if n % bm == 0:
            best = bm
    return best


def _supp_body(x_ref, w1_ref, supp_ref):
    xb = x_ref[...].astype(_BF)
    wb = w1_ref[...].astype(_BF)
    supp_ref[...] = jnp.dot(xb, wb, preferred_element_type=_F32).astype(_BF)


def _pass_a_body(adj_ref, supp_ref, b1_ref, w2_ref, w3_ref, g_ref, adjb_ref):
    ab = adj_ref[...].astype(_BF)
    adjb_ref[...] = ab
    acc = jnp.dot(ab, supp_ref[...], preferred_element_type=_F32)
    h1 = jnp.maximum(acc + b1_ref[...], 0.0)
    w23 = jnp.dot(w2_ref[...].astype(_BF), w3_ref[...].astype(_BF),
                  preferred_element_type=_F32)
    g_ref[...] = jnp.dot(h1.astype(_BF), w23.astype(_BF),
                         preferred_element_type=_F32).astype(_BF)


def _pass_b_body(adjb_ref, g_ref, yaug_ref):
    y = jnp.dot(adjb_ref[...], g_ref[...], preferred_element_type=_F32)
    yaug_ref[...] = jnp.concatenate(
        [y, jnp.ones_like(y)], axis=1).astype(_BF)


def _pass_c_body(adjb_ref, yaug_ref, b2_ref, w3_ref, b3_ref, out_ref):
    c = b3_ref.shape[1]
    p = jnp.dot(adjb_ref[...], yaug_ref[...], preferred_element_type=_F32)
    b23 = jnp.dot(b2_ref[...].astype(_BF), w3_ref[...].astype(_BF),
                  preferred_element_type=_F32)
    t = p[:, :c] + p[:, c:] * b23 + b3_ref[...]
    m = jnp.max(t, axis=1, keepdims=True)
    lse = jnp.log(jnp.sum(jnp.exp(t - m), axis=1, keepdims=True)) + m
    out_ref[...] = t - lse


def kernel(x, adj, W1, b1, W2, b2, W3, b3):
    n, f = x.shape
    h = W1.shape[1]
    c = W3.shape[1]
    bm = _pick_bm(n)
    nb = n // bm
    b1r = b1.reshape(1, h)
    b2r = b2.reshape(1, h)
    b3r = b3.reshape(1, c)

    supp = pl.pallas_call(
        _supp_body,
        grid=(1,),
        in_specs=[
            pl.BlockSpec((n, f), lambda i: (0, 0)),
            pl.BlockSpec((f, h), lambda i: (0, 0)),
        ],
        out_specs=pl.BlockSpec((n, h), lambda i: (0, 0)),
        out_shape=jax.ShapeDtypeStruct((n, h), _BF),
    )(x, W1)

    g, adjb = pl.pallas_call(
        _pass_a_body,
        grid=(nb,),
        in_specs=[
            pl.BlockSpec((bm, n), lambda i: (i, 0)),
            pl.BlockSpec((n, h), lambda i: (0, 0)),
            pl.BlockSpec((1, h), lambda i: (0, 0)),
            pl.BlockSpec((h, h), lambda i: (0, 0)),
            pl.BlockSpec((h, c), lambda i: (0, 0)),
        ],
        out_specs=[
            pl.BlockSpec((bm, c), lambda i: (i, 0)),
            pl.BlockSpec((bm, n), lambda i: (i, 0)),
        ],
        out_shape=[
            jax.ShapeDtypeStruct((n, c), _BF),
            jax.ShapeDtypeStruct((n, n), _BF),
        ],
        compiler_params=pltpu.CompilerParams(
            dimension_semantics=("arbitrary",),
            vmem_limit_bytes=56 * 1024 * 1024,
        ),
    )(adj, supp, b1r, W2, W3)

    yaug = pl.pallas_call(
        _pass_b_body,
        grid=(nb,),
        in_specs=[
            pl.BlockSpec((bm, n), lambda i: (i, 0)),
            pl.BlockSpec((n, c), lambda i: (0, 0)),
        ],
        out_specs=pl.BlockSpec((bm, 2 * c), lambda i: (i, 0)),
        out_shape=jax.ShapeDtypeStruct((n, 2 * c), _BF),
        compiler_params=pltpu.CompilerParams(
            dimension_semantics=("arbitrary",),
            vmem_limit_bytes=56 * 1024 * 1024,
        ),
    )(adjb, g)

    out = pl.pallas_call(
        _pass_c_body,
        grid=(nb,),
        in_specs=[
            pl.BlockSpec((bm, n), lambda i: (i, 0)),
            pl.BlockSpec((n, 2 * c), lambda i: (0, 0)),
            pl.BlockSpec((1, h), lambda i: (0, 0)),
            pl.BlockSpec((h, c), lambda i: (0, 0)),
            pl.BlockSpec((1, c), lambda i: (0, 0)),
        ],
        out_specs=pl.BlockSpec((bm, c), lambda i: (i, 0)),
        out_shape=jax.ShapeDtypeStruct((n, c), _F32),
        compiler_params=pltpu.CompilerParams(
            dimension_semantics=("arbitrary",),
            vmem_limit_bytes=56 * 1024 * 1024,
        ),
    )(adjb, yaug, b2r, W3, b3r)

    return out


# speedup vs baseline: 1.1768x; 1.1567x over previous
import jax
import jax.numpy as jnp
from jax.experimental import pallas as pl
from jax.experimental.pallas import tpu as pltpu

_BF = jnp.bfloat16
_F8 = jnp.float8_e4m3fn
_F32 = jnp.float32


def _pick_bm(n: int, cap: int = 256) -> int:
    best = 8
    for bm in range(8, min(n, cap) + 1, 8):
        if n % bm == 0:
            best = bm
    return best


def _supp_body(x_ref, w1_ref, supp_ref):
    xb = x_ref[...].astype(_BF)
    wb = w1_ref[...].astype(_BF)
    supp_ref[...] = jnp.dot(xb, wb, preferred_element_type=_F32).astype(_BF)


def _pass_a_body(adj_ref, supp_ref, b1_ref, w2_ref, w3_ref, g_ref, adjb_ref):
    a32 = adj_ref[...]
    adjb_ref[...] = a32.astype(_F8)
    acc = jnp.dot(a32.astype(_BF), supp_ref[...], preferred_element_type=_F32)
    h1 = jnp.maximum(acc + b1_ref[...], 0.0)
    w23 = jnp.dot(w2_ref[...].astype(_BF), w3_ref[...].astype(_BF),
                  preferred_element_type=_F32)
    g_ref[...] = jnp.dot(h1.astype(_BF), w23.astype(_BF),
                         preferred_element_type=_F32).astype(_BF)


def _pass_b_body(adjb_ref, g_ref, yaug_ref):
    ab = adjb_ref[...].astype(_BF)
    y = jnp.dot(ab, g_ref[...], preferred_element_type=_F32)
    yaug_ref[...] = jnp.concatenate(
        [y, jnp.ones_like(y)], axis=1).astype(_BF)


def _pass_c_body(adjb_ref, yaug_ref, b2_ref, w3_ref, b3_ref, out_ref):
    c = b3_ref.shape[1]
    ab = adjb_ref[...].astype(_BF)
    p = jnp.dot(ab, yaug_ref[...], preferred_element_type=_F32)
    b23 = jnp.dot(b2_ref[...].astype(_BF), w3_ref[...].astype(_BF),
                  preferred_element_type=_F32)
    t = p[:, :c] + p[:, c:] * b23 + b3_ref[...]
    m = jnp.max(t, axis=1, keepdims=True)
    lse = jnp.log(jnp.sum(jnp.exp(t - m), axis=1, keepdims=True)) + m
    out_ref[...] = t - lse


def kernel(x, adj, W1, b1, W2, b2, W3, b3):
    n, f = x.shape
    h = W1.shape[1]
    c = W3.shape[1]
    bm = _pick_bm(n)
    nb = n // bm
    b1r = b1.reshape(1, h)
    b2r = b2.reshape(1, h)
    b3r = b3.reshape(1, c)

    supp = pl.pallas_call(
        _supp_body,
        grid=(1,),
        in_specs=[
            pl.BlockSpec((n, f), lambda i: (0, 0)),
            pl.BlockSpec((f, h), lambda i: (0, 0)),
        ],
        out_specs=pl.BlockSpec((n, h), lambda i: (0, 0)),
        out_shape=jax.ShapeDtypeStruct((n, h), _BF),
    )(x, W1)

    g, adjb = pl.pallas_call(
        _pass_a_body,
        grid=(nb,),
        in_specs=[
            pl.BlockSpec((bm, n), lambda i: (i, 0)),
            pl.BlockSpec((n, h), lambda i: (0, 0)),
            pl.BlockSpec((1, h), lambda i: (0, 0)),
            pl.BlockSpec((h, h), lambda i: (0, 0)),
            pl.BlockSpec((h, c), lambda i: (0, 0)),
        ],
        out_specs=[
            pl.BlockSpec((bm, c), lambda i: (i, 0)),
            pl.BlockSpec((bm, n), lambda i: (i, 0)),
        ],
        out_shape=[
            jax.ShapeDtypeStruct((n, c), _BF),
            jax.ShapeDtypeStruct((n, n), _F8),
        ],
        compiler_params=pltpu.CompilerParams(
            dimension_semantics=("arbitrary",),
            vmem_limit_bytes=56 * 1024 * 1024,
        ),
    )(adj, supp, b1r, W2, W3)

    yaug = pl.pallas_call(
        _pass_b_body,
        grid=(nb,),
        in_specs=[
            pl.BlockSpec((bm, n), lambda i: (i, 0)),
            pl.BlockSpec((n, c), lambda i: (0, 0)),
        ],
        out_specs=pl.BlockSpec((bm, 2 * c), lambda i: (i, 0)),
        out_shape=jax.ShapeDtypeStruct((n, 2 * c), _BF),
        compiler_params=pltpu.CompilerParams(
            dimension_semantics=("arbitrary",),
            vmem_limit_bytes=56 * 1024 * 1024,
        ),
    )(adjb, g)

    out = pl.pallas_call(
        _pass_c_body,
        grid=(nb,),
        in_specs=[
            pl.BlockSpec((bm, n), lambda i: (i, 0)),
            pl.BlockSpec((n, 2 * c), lambda i: (0, 0)),
            pl.BlockSpec((1, h), lambda i: (0, 0)),
            pl.BlockSpec((h, c), lambda i: (0, 0)),
            pl.BlockSpec((1, c), lambda i: (0, 0)),
        ],
        out_specs=pl.BlockSpec((bm, c), lambda i: (i, 0)),
        out_shape=jax.ShapeDtypeStruct((n, c), _F32),
        compiler_params=pltpu.CompilerParams(
            dimension_semantics=("arbitrary",),
            vmem_limit_bytes=56 * 1024 * 1024,
        ),
    )(adjb, yaug, b2r, W3, b3r)

    return out


# speedup vs baseline: 1.1775x; 1.0006x over previous
import jax
import jax.numpy as jnp
from jax.experimental import pallas as pl
from jax.experimental.pallas import tpu as pltpu

_BF = jnp.bfloat16
_F8 = jnp.float8_e4m3fn
_F32 = jnp.float32


def _pick_bm(n: int, cap: int = 256) -> int:
    best = 8
    for bm in range(8, min(n, cap) + 1, 8):
        if n % bm == 0:
            best = bm
    return best


def _supp_body(x_ref, w1_ref, supp_ref):
    xb = x_ref[...].astype(_BF)
    wb = w1_ref[...].astype(_BF)
    supp_ref[...] = jnp.dot(xb, wb, preferred_element_type=_F32).astype(_BF)


def _pass_a_body(adj_ref, supp_ref, b1_ref, w2_ref, w3_ref, g_ref, adjb_ref):
    a32 = adj_ref[...]
    adjb_ref[...] = a32.astype(_F8)
    acc = jnp.dot(a32.astype(_BF), supp_ref[...], preferred_element_type=_F32)
    h1 = jnp.maximum(acc + b1_ref[...], 0.0)
    w23 = jnp.dot(w2_ref[...].astype(_BF), w3_ref[...].astype(_BF),
                  preferred_element_type=_F32)
    g_ref[...] = jnp.dot(h1.astype(_BF), w23.astype(_BF),
                         preferred_element_type=_F32).astype(_BF)


def _pass_b_body(adjb_ref, g_ref, yaug_ref):
    ab = adjb_ref[...].astype(_BF)
    y = jnp.dot(ab, g_ref[...], preferred_element_type=_F32)
    yaug_ref[...] = jnp.concatenate(
        [y, jnp.ones_like(y)], axis=1).astype(_BF)


def _pass_c_body(adjb_ref, yaug_ref, b2_ref, w3_ref, b3_ref, out_ref):
    c = b3_ref.shape[1]
    ab = adjb_ref[...].astype(_BF)
    p = jnp.dot(ab, yaug_ref[...], preferred_element_type=_F32)
    b23 = jnp.dot(b2_ref[...].astype(_BF), w3_ref[...].astype(_BF),
                  preferred_element_type=_F32)
    t = p[:, :c] + p[:, c:] * b23 + b3_ref[...]
    m = jnp.max(t, axis=1, keepdims=True)
    lse = jnp.log(jnp.sum(jnp.exp(t - m), axis=1, keepdims=True)) + m
    out_ref[...] = t - lse


def kernel(x, adj, W1, b1, W2, b2, W3, b3):
    n, f = x.shape
    h = W1.shape[1]
    c = W3.shape[1]
    bm = _pick_bm(n)
    nb = n // bm
    b1r = b1.reshape(1, h)
    b2r = b2.reshape(1, h)
    b3r = b3.reshape(1, c)

    supp = pl.pallas_call(
        _supp_body,
        grid=(1,),
        in_specs=[
            pl.BlockSpec((n, f), lambda i: (0, 0)),
            pl.BlockSpec((f, h), lambda i: (0, 0)),
        ],
        out_specs=pl.BlockSpec((n, h), lambda i: (0, 0)),
        out_shape=jax.ShapeDtypeStruct((n, h), _BF),
    )(x, W1)

    g, adjb = pl.pallas_call(
        _pass_a_body,
        grid=(nb,),
        in_specs=[
            pl.BlockSpec((bm, n), lambda i: (i, 0)),
            pl.BlockSpec((n, h), lambda i: (0, 0)),
            pl.BlockSpec((1, h), lambda i: (0, 0)),
            pl.BlockSpec((h, h), lambda i: (0, 0)),
            pl.BlockSpec((h, c), lambda i: (0, 0)),
        ],
        out_specs=[
            pl.BlockSpec((bm, c), lambda i: (i, 0)),
            pl.BlockSpec((bm, n), lambda i: (i, 0)),
        ],
        out_shape=[
            jax.ShapeDtypeStruct((n, c), _BF),
            jax.ShapeDtypeStruct((n, n), _F8),
        ],
        compiler_params=pltpu.CompilerParams(
            dimension_semantics=("parallel",),
            vmem_limit_bytes=56 * 1024 * 1024,
        ),
    )(adj, supp, b1r, W2, W3)

    yaug = pl.pallas_call(
        _pass_b_body,
        grid=(nb,),
        in_specs=[
            pl.BlockSpec((bm, n), lambda i: (i, 0)),
            pl.BlockSpec((n, c), lambda i: (0, 0)),
        ],
        out_specs=pl.BlockSpec((bm, 2 * c), lambda i: (i, 0)),
        out_shape=jax.ShapeDtypeStruct((n, 2 * c), _BF),
        compiler_params=pltpu.CompilerParams(
            dimension_semantics=("parallel",),
            vmem_limit_bytes=56 * 1024 * 1024,
        ),
    )(adjb, g)

    out = pl.pallas_call(
        _pass_c_body,
        grid=(nb,),
        in_specs=[
            pl.BlockSpec((bm, n), lambda i: (i, 0)),
            pl.BlockSpec((n, 2 * c), lambda i: (0, 0)),
            pl.BlockSpec((1, h), lambda i: (0, 0)),
            pl.BlockSpec((h, c), lambda i: (0, 0)),
            pl.BlockSpec((1, c), lambda i: (0, 0)),
        ],
        out_specs=pl.BlockSpec((bm, c), lambda i: (i, 0)),
        out_shape=jax.ShapeDtypeStruct((n, c), _F32),
        compiler_params=pltpu.CompilerParams(
            dimension_semantics=("parallel",),
            vmem_limit_bytes=56 * 1024 * 1024,
        ),
    )(adjb, yaug, b2r, W3, b3r)

    return out


# speedup vs baseline: 1.4375x; 1.2208x over previous
import jax
import jax.numpy as jnp
from jax.experimental import pallas as pl
from jax.experimental.pallas import tpu as pltpu

_BF = jnp.bfloat16
_F8 = jnp.float8_e4m3fn
_F32 = jnp.float32

_YSCALE = 2.0 ** -11
_VMEM = 58 * 1024 * 1024


def _pick_bm(n: int, cap: int) -> int:
    best = 8
    for bm in range(8, min(n, cap) + 1, 8):
        if n % bm == 0:
            best = bm
    return best


def _supp_body(x_ref, w1_ref, supp_ref):
    xb = x_ref[...].astype(_BF)
    wb = w1_ref[...].astype(_BF)
    s = jnp.dot(xb, wb, preferred_element_type=_F32).astype(_BF)
    nones = supp_ref.shape[1] - w1_ref.shape[1]
    ones = jnp.ones(s.shape[:1] + (nones,), _BF)
    supp_ref[...] = jnp.concatenate([s, ones], axis=1)


def _pass_a_body(adj_ref, supp_ref, b1_ref, w2_ref, w3_ref,
                 g_ref, rs_ref, adjb_ref):
    h = b1_ref.shape[1]
    a32 = adj_ref[...]
    adjb_ref[...] = a32.astype(_F8)
    acc = jnp.dot(a32.astype(_BF), supp_ref[...], preferred_element_type=_F32)
    h1 = jnp.maximum(acc[:, :h] + b1_ref[...], 0.0)
    rs_ref[...] = acc[:, h:]
    w23 = jnp.dot(w2_ref[...].astype(_BF), w3_ref[...].astype(_BF),
                  preferred_element_type=_F32)
    g_ref[...] = jnp.dot(h1.astype(_BF), w23.astype(_BF),
                         preferred_element_type=_F32).astype(_F8)


def _pass_b_body(adjb_ref, g_ref, y_ref):
    y = jnp.dot(adjb_ref[...], g_ref[...], preferred_element_type=_F32)
    y_ref[...] = (y * _YSCALE).astype(_F8)


def _pass_c_body(adjb_ref, y_ref, rs_ref, b2_ref, w3_ref, b3_ref, out_ref):
    p = jnp.dot(adjb_ref[...], y_ref[...],
                preferred_element_type=_F32) * (1.0 / _YSCALE)
    b23 = jnp.dot(b2_ref[...].astype(_BF), w3_ref[...].astype(_BF),
                  preferred_element_type=_F32)
    t = p + rs_ref[...] * b23 + b3_ref[...]
    m = jnp.max(t, axis=1, keepdims=True)
    lse = jnp.log(jnp.sum(jnp.exp(t - m), axis=1, keepdims=True)) + m
    out_ref[...] = t - lse


def kernel(x, adj, W1, b1, W2, b2, W3, b3):
    n, f = x.shape
    h = W1.shape[1]
    c = W3.shape[1]
    bm = _pick_bm(n, 400)
    nb = n // bm
    b1r = b1.reshape(1, h)
    b2r = b2.reshape(1, h)
    b3r = b3.reshape(1, c)

    supp = pl.pallas_call(
        _supp_body,
        grid=(1,),
        in_specs=[
            pl.BlockSpec((n, f), lambda i: (0, 0)),
            pl.BlockSpec((f, h), lambda i: (0, 0)),
        ],
        out_specs=pl.BlockSpec((n, h + c), lambda i: (0, 0)),
        out_shape=jax.ShapeDtypeStruct((n, h + c), _BF),
    )(x, W1)

    g, rs, adjb = pl.pallas_call(
        _pass_a_body,
        grid=(nb,),
        in_specs=[
            pl.BlockSpec((bm, n), lambda i: (i, 0)),
            pl.BlockSpec((n, h + c), lambda i: (0, 0)),
            pl.BlockSpec((1, h), lambda i: (0, 0)),
            pl.BlockSpec((h, h), lambda i: (0, 0)),
            pl.BlockSpec((h, c), lambda i: (0, 0)),
        ],
        out_specs=[
            pl.BlockSpec((bm, c), lambda i: (i, 0)),
            pl.BlockSpec((bm, c), lambda i: (i, 0)),
            pl.BlockSpec((bm, n), lambda i: (i, 0)),
        ],
        out_shape=[
            jax.ShapeDtypeStruct((n, c), _F8),
            jax.ShapeDtypeStruct((n, c), _F32),
            jax.ShapeDtypeStruct((n, n), _F8),
        ],
        compiler_params=pltpu.CompilerParams(
            dimension_semantics=("arbitrary",),
            vmem_limit_bytes=_VMEM,
        ),
    )(adj, supp, b1r, W2, W3)

    y = pl.pallas_call(
        _pass_b_body,
        grid=(nb,),
        in_specs=[
            pl.BlockSpec((bm, n), lambda i: (i, 0)),
            pl.BlockSpec((n, c), lambda i: (0, 0)),
        ],
        out_specs=pl.BlockSpec((bm, c), lambda i: (i, 0)),
        out_shape=jax.ShapeDtypeStruct((n, c), _F8),
        compiler_params=pltpu.CompilerParams(
            dimension_semantics=("arbitrary",),
            vmem_limit_bytes=_VMEM,
        ),
    )(adjb, g)

    out = pl.pallas_call(
        _pass_c_body,
        grid=(nb,),
        in_specs=[
            pl.BlockSpec((bm, n), lambda i: (i, 0)),
            pl.BlockSpec((n, c), lambda i: (0, 0)),
            pl.BlockSpec((bm, c), lambda i: (i, 0)),
            pl.BlockSpec((1, h), lambda i: (0, 0)),
            pl.BlockSpec((h, c), lambda i: (0, 0)),
            pl.BlockSpec((1, c), lambda i: (0, 0)),
        ],
        out_specs=pl.BlockSpec((bm, c), lambda i: (i, 0)),
        out_shape=jax.ShapeDtypeStruct((n, c), _F32),
        compiler_params=pltpu.CompilerParams(
            dimension_semantics=("arbitrary",),
            vmem_limit_bytes=_VMEM,
        ),
    )(adjb, y, rs, b2r, W3, b3r)

    return out
